# baseline (speedup 1.0000x reference)
import functools

import jax
import jax.numpy as jnp
from jax import lax
from jax.experimental import pallas as pl
from jax.experimental.pallas import tpu as pltpu
from jax.experimental.pallas import tpu_sc as plsc

N = 10000
E = 640000
D = 128
NTYPES = 9
NSTEPS = 10
EPS = 1e-5

NC = 2
NS = 16
NW = NC * NS
NSLAB = 2
E2 = E // NSLAB
EPW = E2 // NW
CHUNK = 80
NCH = EPW // CHUNK
NPAD = 10240
RPT = NPAD // NS

BN = 1000
BE = 2000


def _ln(h, g, b):
    m = jnp.mean(h, axis=1, keepdims=True)
    v = jnp.mean((h - m) * (h - m), axis=1, keepdims=True)
    return (h - m) * lax.rsqrt(v + EPS) * g + b


def _mm(a, b):
    return jnp.dot(a, b, preferred_element_type=jnp.float32)


def _node_encode_body(vel, nt, w1v, w1o, b1, w2, b2, w3, b3, lg, lb, wa, wb,
                      x_o, u_o, v_o):
    oh = (nt[...] == lax.broadcasted_iota(jnp.int32, (BN, NTYPES), 1)
          ).astype(jnp.float32)
    h = jnp.maximum(_mm(vel[...], w1v[...]) + _mm(oh, w1o[...]) + b1[...], 0.0)
    h = jnp.maximum(_mm(h, w2[...]) + b2[...], 0.0)
    h = _mm(h, w3[...]) + b3[...]
    x = _ln(h, lg[...], lb[...])
    x_o[...] = x
    u_o[...] = _mm(x, wa[...])
    v_o[...] = _mm(x, wb[...])


def _edge_encode_body(ef, w1, b1, w2, b2, w3, b3, lg, lb, e_o):
    h = jnp.maximum(_mm(ef[...], w1[...]) + b1[...], 0.0)
    h = jnp.maximum(_mm(h, w2[...]) + b2[...], 0.0)
    h = _mm(h, w3[...]) + b3[...]
    e_o[...] = _ln(h, lg[...], lb[...])


def _edge_step_body(g, e, w1c, b1, w2, b2, w3, b3, lg, lb, e_o, enew_o):
    e_in = e[...]
    h = jnp.maximum(g[...] + _mm(e_in, w1c[...]) + b1[...], 0.0)
    h = jnp.maximum(_mm(h, w2[...]) + b2[...], 0.0)
    h = _mm(h, w3[...]) + b3[...]
    enew = _ln(h, lg[...], lb[...])
    enew_o[...] = enew
    e_o[...] = e_in + enew


def _sum_aggs(aggs):
    acc = aggs[0][0] + aggs[0][1]
    for a in aggs[1:]:
        acc = acc + (a[0] + a[1])
    return acc


def _node_step_body(*refs):
    x, aggs, rest = refs[0], refs[1:1 + NSLAB], refs[1 + NSLAB:]
    w1x, w1a, b1, w2, b2, w3, b3, lg, lb, wa, wb, x_o, u_o, v_o = rest
    x_in = x[...]
    agg = _sum_aggs(aggs)
    h = jnp.maximum(_mm(x_in, w1x[...]) + _mm(agg, w1a[...]) + b1[...], 0.0)
    h = jnp.maximum(_mm(h, w2[...]) + b2[...], 0.0)
    h = _mm(h, w3[...]) + b3[...]
    xn = x_in + _ln(h, lg[...], lb[...])
    x_o[...] = xn
    u_o[...] = _mm(xn, wa[...])
    v_o[...] = _mm(xn, wb[...])


def _final_body(*refs):
    x, aggs, rest = refs[0], refs[1:1 + NSLAB], refs[1 + NSLAB:]
    w1x, w1a, b1, w2, b2, w3, b3, lg, lb, dw1, db1, dw2, db2, dw3, db3, out_o = rest
    x_in = x[...]
    agg = _sum_aggs(aggs)
    h = jnp.maximum(_mm(x_in, w1x[...]) + _mm(agg, w1a[...]) + b1[...], 0.0)
    h = jnp.maximum(_mm(h, w2[...]) + b2[...], 0.0)
    h = _mm(h, w3[...]) + b3[...]
    xn = x_in + _ln(h, lg[...], lb[...])
    h = jnp.maximum(_mm(xn, dw1[...]) + db1[...], 0.0)
    h = jnp.maximum(_mm(h, dw2[...]) + db2[...], 0.0)
    out_o[...] = _mm(h, dw3[...]) + db3[...]


def _row_spec(bn, width):
    return pl.BlockSpec((bn, width), lambda i: (i, 0))


def _full_spec(shape):
    nd = len(shape)
    return pl.BlockSpec(shape, lambda i: (0,) * nd)


def _wspecs(shapes):
    return [_full_spec(s) for s in shapes]


_SC_MESH = plsc.VectorSubcoreMesh(core_axis_name="c", subcore_axis_name="s",
                                  num_cores=NC, num_subcores=NS)

NBUF = 5
NGRP = NCH // NBUF


@functools.partial(
    pl.kernel,
    out_type=jax.ShapeDtypeStruct((E2, D), jnp.float32),
    mesh=_SC_MESH,
    scratch_types=[
        pltpu.VMEM((EPW,), jnp.int32),
        pltpu.VMEM((EPW,), jnp.int32),
        pltpu.VMEM((NBUF, CHUNK, D), jnp.float32),
        pltpu.SemaphoreType.DMA((NBUF,)),
        pltpu.SemaphoreType.DMA((NBUF,)),
        pltpu.SemaphoreType.DMA((NBUF,)),
    ],
)
def _gather_g(u_hbm, v_hbm, src_hbm, dst_hbm, g_hbm, sidx, didx, rows,
              sem_u, sem_v, sem_d):
    wid = lax.axis_index("s") * NC + lax.axis_index("c")
    base = wid * EPW
    pltpu.sync_copy(src_hbm.at[pl.ds(base, EPW)], sidx)
    pltpu.sync_copy(dst_hbm.at[pl.ds(base, EPW)], didx)

    def group(g, carry):
        for b in range(NBUF):
            @pl.when(g > 0)
            def _():
                pltpu.make_async_copy(
                    rows.at[b], g_hbm.at[pl.ds(base, CHUNK), :], sem_d.at[b]
                ).wait()
        du = []
        for b in range(NBUF):
            j = g * NBUF + b
            du.append(pltpu.async_copy(
                u_hbm.at[sidx.at[pl.ds(j * CHUNK, CHUNK)]], rows.at[b],
                sem_u.at[b]))
        dv = []
        for b in range(NBUF):
            j = g * NBUF + b
            du[b].wait()
            dv.append(pltpu.async_copy(
                v_hbm.at[didx.at[pl.ds(j * CHUNK, CHUNK)]], rows.at[b],
                sem_v.at[b], add=True))
        for b in range(NBUF):
            j = g * NBUF + b
            dv[b].wait()
            pltpu.async_copy(rows.at[b],
                             g_hbm.at[pl.ds(base + j * CHUNK, CHUNK), :],
                             sem_d.at[b])
        return carry

    lax.fori_loop(0, NGRP, group, 0)
    for b in range(NBUF):
        pltpu.make_async_copy(
            rows.at[b], g_hbm.at[pl.ds(base, CHUNK), :], sem_d.at[b]).wait()


CS = 40
NCHS = EPW // CS
NGRPS = NCHS // NBUF


@functools.partial(
    pl.kernel,
    out_type=jax.ShapeDtypeStruct((NC, NPAD, D), jnp.float32),
    mesh=_SC_MESH,
    scratch_types=[
        pltpu.VMEM((NBUF, CS), jnp.int32),
        pltpu.VMEM((NBUF, CS, D), jnp.float32),
        pltpu.VMEM_SHARED((NPAD, D), jnp.float32),
        pltpu.SemaphoreType.DMA((NBUF,)),
        pltpu.SemaphoreType.DMA((NBUF,)),
        pltpu.SemaphoreType.DMA((NBUF,)),
    ],
)
def _scatter_agg(enew_hbm, dst_hbm, out_hbm, idx2, rows, agg_sh,
                 sem_a, sem_b, sem_c):
    cid = lax.axis_index("c")
    sid = lax.axis_index("s")
    wid = sid * NC + cid
    base = wid * EPW

    def zr(r, carry):
        def zl(l, c2):
            rows[0, r, pl.ds(l * 16, 16)] = jnp.zeros((16,), jnp.float32)
            return c2
        return lax.fori_loop(0, 8, zl, carry)

    lax.fori_loop(0, CS, zr, 0)
    for k in range(RPT // CS):
        pltpu.sync_copy(rows.at[0],
                        agg_sh.at[pl.ds(sid * RPT + k * CS, CS), :])
    plsc.subcore_barrier()

    def group(g, carry):
        for b in range(NBUF):
            @pl.when(g > 0)
            def _():
                pltpu.make_async_copy(
                    rows.at[b], agg_sh.at[idx2.at[b]], sem_c.at[b]).wait()
        da, db = [], []
        for b in range(NBUF):
            j = g * NBUF + b
            off = base + j * CS
            da.append(pltpu.async_copy(
                dst_hbm.at[pl.ds(off, CS)], idx2.at[b], sem_a.at[b]))
            db.append(pltpu.async_copy(
                enew_hbm.at[pl.ds(off, CS), :], rows.at[b], sem_b.at[b]))
        for b in range(NBUF):
            da[b].wait()
            db[b].wait()
            pltpu.async_copy(rows.at[b], agg_sh.at[idx2.at[b]], sem_c.at[b],
                             add=True)
        return carry

    lax.fori_loop(0, NGRPS, group, 0)
    for b in range(NBUF):
        pltpu.make_async_copy(
            rows.at[b], agg_sh.at[idx2.at[b]], sem_c.at[b]).wait()
    plsc.subcore_barrier()

    pltpu.sync_copy(agg_sh.at[pl.ds(sid * RPT, RPT), :],
                    out_hbm.at[cid, pl.ds(sid * RPT, RPT), :])


def _enc_w(enc):
    m = enc["mlp"]
    return (m[0]["w"], m[0]["b"].reshape(1, D),
            m[1]["w"], m[1]["b"].reshape(1, D),
            m[2]["w"], m[2]["b"].reshape(1, D),
            enc["ln_g"].reshape(1, D), enc["ln_b"].reshape(1, D))


def kernel(velocity, node_type, edge_index, edge_features, params):
    src = edge_index[0]
    dst = edge_index[1]
    steps = params["steps"]

    ne_w1, ne_b1, ne_w2, ne_b2, ne_w3, ne_b3, ne_g, ne_b = _enc_w(params["node_enc"])
    wa0 = steps[0]["edge_fn"]["mlp"][0]["w"][0:D]
    wb0 = steps[0]["edge_fn"]["mlp"][0]["w"][D:2 * D]

    x, u, v = pl.pallas_call(
        _node_encode_body,
        grid=(N // BN,),
        in_specs=[_row_spec(BN, 3), _row_spec(BN, 1)] + _wspecs(
            [(3, D), (NTYPES, D), (1, D), (D, D), (1, D), (D, D), (1, D),
             (1, D), (1, D), (D, D), (D, D)]),
        out_specs=[_row_spec(BN, D)] * 3,
        out_shape=[jax.ShapeDtypeStruct((N, D), jnp.float32)] * 3,
    )(velocity, node_type.astype(jnp.int32), ne_w1[0:3], ne_w1[3:12], ne_b1,
      ne_w2, ne_b2, ne_w3, ne_b3, ne_g, ne_b, wa0, wb0)

    ee_w1, ee_b1, ee_w2, ee_b2, ee_w3, ee_b3, ee_g, ee_b = _enc_w(params["edge_enc"])
    srcs = [src[sl * E2:(sl + 1) * E2] for sl in range(NSLAB)]
    dsts = [dst[sl * E2:(sl + 1) * E2] for sl in range(NSLAB)]
    es = [pl.pallas_call(
        _edge_encode_body,
        grid=(E2 // BE,),
        in_specs=[_row_spec(BE, 3)] + _wspecs(
            [(3, D), (1, D), (D, D), (1, D), (D, D), (1, D), (1, D), (1, D)]),
        out_specs=_row_spec(BE, D),
        out_shape=jax.ShapeDtypeStruct((E2, D), jnp.float32),
    )(edge_features[sl * E2:(sl + 1) * E2], ee_w1, ee_b1, ee_w2, ee_b2,
      ee_w3, ee_b3, ee_g, ee_b) for sl in range(NSLAB)]

    for s in range(NSTEPS):
        ef_w1, ef_b1, ef_w2, ef_b2, ef_w3, ef_b3, ef_g, ef_b = _enc_w(steps[s]["edge_fn"])
        aggs = []
        for sl in range(NSLAB):
            g = _gather_g(u, v, srcs[sl], dsts[sl])
            es[sl], enew = pl.pallas_call(
                _edge_step_body,
                grid=(E2 // BE,),
                in_specs=[_row_spec(BE, D), _row_spec(BE, D)] + _wspecs(
                    [(D, D), (1, D), (D, D), (1, D), (D, D), (1, D), (1, D), (1, D)]),
                out_specs=[_row_spec(BE, D)] * 2,
                out_shape=[jax.ShapeDtypeStruct((E2, D), jnp.float32)] * 2,
            )(g, es[sl], ef_w1[2 * D:3 * D], ef_b1, ef_w2, ef_b2, ef_w3,
              ef_b3, ef_g, ef_b)
            aggs.append(_scatter_agg(enew, dsts[sl])[:, :N, :])

        nf_w1, nf_b1, nf_w2, nf_b2, nf_w3, nf_b3, nf_g, nf_b = _enc_w(steps[s]["node_fn"])
        agg_spec = pl.BlockSpec((NC, BN, D), lambda i: (0, i, 0))
        if s < NSTEPS - 1:
            wa = steps[s + 1]["edge_fn"]["mlp"][0]["w"][0:D]
            wb = steps[s + 1]["edge_fn"]["mlp"][0]["w"][D:2 * D]
            x, u, v = pl.pallas_call(
                _node_step_body,
                grid=(N // BN,),
                in_specs=[_row_spec(BN, D)] + [agg_spec] * NSLAB + _wspecs(
                    [(D, D), (D, D), (1, D), (D, D), (1, D), (D, D), (1, D),
                     (1, D), (1, D), (D, D), (D, D)]),
                out_specs=[_row_spec(BN, D)] * 3,
                out_shape=[jax.ShapeDtypeStruct((N, D), jnp.float32)] * 3,
            )(x, *aggs, nf_w1[0:D], nf_w1[D:2 * D], nf_b1, nf_w2,
              nf_b2, nf_w3, nf_b3, nf_g, nf_b, wa, wb)
        else:
            dm = params["decoder"]["mlp"]
            out = pl.pallas_call(
                _final_body,
                grid=(N // BN,),
                in_specs=[_row_spec(BN, D)] + [agg_spec] * NSLAB + _wspecs(
                    [(D, D), (D, D), (1, D), (D, D), (1, D), (D, D), (1, D),
                     (1, D), (1, D),
                     (D, D), (1, D), (D, D), (1, D), (D, 3), (1, 3)]),
                out_specs=_row_spec(BN, 3),
                out_shape=jax.ShapeDtypeStruct((N, 3), jnp.float32),
            )(x, *aggs, nf_w1[0:D], nf_w1[D:2 * D], nf_b1, nf_w2,
              nf_b2, nf_w3, nf_b3, nf_g, nf_b,
              dm[0]["w"], dm[0]["b"].reshape(1, D),
              dm[1]["w"], dm[1]["b"].reshape(1, D),
              dm[2]["w"], dm[2]["b"].reshape(1, 3))
    return out


# speedup vs baseline: 1.0327x; 1.0327x over previous
import functools

import jax
import jax.numpy as jnp
from jax import lax
from jax.experimental import pallas as pl
from jax.experimental.pallas import tpu as pltpu
from jax.experimental.pallas import tpu_sc as plsc

N = 10000
E = 640000
D = 128
NTYPES = 9
NSTEPS = 10
EPS = 1e-5

NC = 2
NS = 16
NW = NC * NS
NSLAB = 2
E2 = E // NSLAB
EPW = E2 // NW
CHUNK = 80
NCH = EPW // CHUNK
NPAD = 10240
RPT = NPAD // NS

BN = 1000
BE = 2560


def _ln(h, g, b):
    m = jnp.mean(h, axis=1, keepdims=True)
    v = jnp.mean((h - m) * (h - m), axis=1, keepdims=True)
    return (h - m) * lax.rsqrt(v + EPS) * g + b


def _mm(a, b):
    return jnp.dot(a, b, preferred_element_type=jnp.float32)


def _node_encode_body(vel, nt, w1v, w1o, b1, w2, b2, w3, b3, lg, lb, wa, wb,
                      x_o, u_o, v_o):
    oh = (nt[...] == lax.broadcasted_iota(jnp.int32, (BN, NTYPES), 1)
          ).astype(jnp.float32)
    h = jnp.maximum(_mm(vel[...], w1v[...]) + _mm(oh, w1o[...]) + b1[...], 0.0)
    h = jnp.maximum(_mm(h, w2[...]) + b2[...], 0.0)
    h = _mm(h, w3[...]) + b3[...]
    x = _ln(h, lg[...], lb[...])
    x_o[...] = x
    u_o[...] = _mm(x, wa[...])
    v_o[...] = _mm(x, wb[...])


def _edge_encode_body(ef, w1, b1, w2, b2, w3, b3, lg, lb, e_o):
    h = jnp.maximum(_mm(ef[...], w1[...]) + b1[...], 0.0)
    h = jnp.maximum(_mm(h, w2[...]) + b2[...], 0.0)
    h = _mm(h, w3[...]) + b3[...]
    e_o[...] = _ln(h, lg[...], lb[...])


def _edge_step_body(g, e, w1c, b1, w2, b2, w3, b3, lg, lb, e_o, enew_o):
    e_in = e[...]
    h = jnp.maximum(g[...] + _mm(e_in, w1c[...]) + b1[...], 0.0)
    h = jnp.maximum(_mm(h, w2[...]) + b2[...], 0.0)
    h = _mm(h, w3[...]) + b3[...]
    enew = _ln(h, lg[...], lb[...])
    enew_o[...] = enew
    e_o[...] = e_in + enew


def _sum_aggs(aggs):
    acc = aggs[0][0] + aggs[0][1]
    for a in aggs[1:]:
        acc = acc + (a[0] + a[1])
    return acc


def _node_step_body(*refs):
    x, aggs, rest = refs[0], refs[1:1 + NSLAB], refs[1 + NSLAB:]
    w1x, w1a, b1, w2, b2, w3, b3, lg, lb, wa, wb, x_o, u_o, v_o = rest
    x_in = x[...]
    agg = _sum_aggs(aggs)
    h = jnp.maximum(_mm(x_in, w1x[...]) + _mm(agg, w1a[...]) + b1[...], 0.0)
    h = jnp.maximum(_mm(h, w2[...]) + b2[...], 0.0)
    h = _mm(h, w3[...]) + b3[...]
    xn = x_in + _ln(h, lg[...], lb[...])
    x_o[...] = xn
    u_o[...] = _mm(xn, wa[...])
    v_o[...] = _mm(xn, wb[...])


def _final_body(*refs):
    x, aggs, rest = refs[0], refs[1:1 + NSLAB], refs[1 + NSLAB:]
    w1x, w1a, b1, w2, b2, w3, b3, lg, lb, dw1, db1, dw2, db2, dw3, db3, out_o = rest
    x_in = x[...]
    agg = _sum_aggs(aggs)
    h = jnp.maximum(_mm(x_in, w1x[...]) + _mm(agg, w1a[...]) + b1[...], 0.0)
    h = jnp.maximum(_mm(h, w2[...]) + b2[...], 0.0)
    h = _mm(h, w3[...]) + b3[...]
    xn = x_in + _ln(h, lg[...], lb[...])
    h = jnp.maximum(_mm(xn, dw1[...]) + db1[...], 0.0)
    h = jnp.maximum(_mm(h, dw2[...]) + db2[...], 0.0)
    out_o[...] = _mm(h, dw3[...]) + db3[...]


def _row_spec(bn, width):
    return pl.BlockSpec((bn, width), lambda i: (i, 0))


def _full_spec(shape):
    nd = len(shape)
    return pl.BlockSpec(shape, lambda i: (0,) * nd)


def _wspecs(shapes):
    return [_full_spec(s) for s in shapes]


_SC_MESH = plsc.VectorSubcoreMesh(core_axis_name="c", subcore_axis_name="s",
                                  num_cores=NC, num_subcores=NS)

NBUF = 5
NGRP = NCH // NBUF


@functools.partial(
    pl.kernel,
    out_type=jax.ShapeDtypeStruct((E2, D), jnp.float32),
    mesh=_SC_MESH,
    scratch_types=[
        pltpu.VMEM((EPW,), jnp.int32),
        pltpu.VMEM((EPW,), jnp.int32),
        pltpu.VMEM((NBUF, CHUNK, D), jnp.float32),
        pltpu.SemaphoreType.DMA((NBUF,)),
        pltpu.SemaphoreType.DMA((NBUF,)),
        pltpu.SemaphoreType.DMA((NBUF,)),
    ],
)
def _gather_g(u_hbm, v_hbm, src_hbm, dst_hbm, g_hbm, sidx, didx, rows,
              sem_u, sem_v, sem_d):
    wid = lax.axis_index("s") * NC + lax.axis_index("c")
    base = wid * EPW
    pltpu.sync_copy(src_hbm.at[pl.ds(base, EPW)], sidx)
    pltpu.sync_copy(dst_hbm.at[pl.ds(base, EPW)], didx)

    def group(g, carry):
        for b in range(NBUF):
            @pl.when(g > 0)
            def _():
                pltpu.make_async_copy(
                    rows.at[b], g_hbm.at[pl.ds(base, CHUNK), :], sem_d.at[b]
                ).wait()
        du = []
        for b in range(NBUF):
            j = g * NBUF + b
            du.append(pltpu.async_copy(
                u_hbm.at[sidx.at[pl.ds(j * CHUNK, CHUNK)]], rows.at[b],
                sem_u.at[b]))
        dv = []
        for b in range(NBUF):
            j = g * NBUF + b
            du[b].wait()
            dv.append(pltpu.async_copy(
                v_hbm.at[didx.at[pl.ds(j * CHUNK, CHUNK)]], rows.at[b],
                sem_v.at[b], add=True))
        for b in range(NBUF):
            j = g * NBUF + b
            dv[b].wait()
            pltpu.async_copy(rows.at[b],
                             g_hbm.at[pl.ds(base + j * CHUNK, CHUNK), :],
                             sem_d.at[b])
        return carry

    lax.fori_loop(0, NGRP, group, 0)
    for b in range(NBUF):
        pltpu.make_async_copy(
            rows.at[b], g_hbm.at[pl.ds(base, CHUNK), :], sem_d.at[b]).wait()


CS = 40
NCHS = EPW // CS
NGRPS = NCHS // NBUF


@functools.partial(
    pl.kernel,
    out_type=jax.ShapeDtypeStruct((NC, NPAD, D), jnp.float32),
    mesh=_SC_MESH,
    scratch_types=[
        pltpu.VMEM((NBUF, CS), jnp.int32),
        pltpu.VMEM((NBUF, CS, D), jnp.float32),
        pltpu.VMEM_SHARED((NPAD, D), jnp.float32),
        pltpu.SemaphoreType.DMA((NBUF,)),
        pltpu.SemaphoreType.DMA((NBUF,)),
        pltpu.SemaphoreType.DMA((NBUF,)),
    ],
)
def _scatter_agg(enew_hbm, dst_hbm, out_hbm, idx2, rows, agg_sh,
                 sem_a, sem_b, sem_c):
    cid = lax.axis_index("c")
    sid = lax.axis_index("s")
    wid = sid * NC + cid
    base = wid * EPW

    def zr(r, carry):
        def zl(l, c2):
            rows[0, r, pl.ds(l * 16, 16)] = jnp.zeros((16,), jnp.float32)
            return c2
        return lax.fori_loop(0, 8, zl, carry)

    lax.fori_loop(0, CS, zr, 0)
    for k in range(RPT // CS):
        pltpu.sync_copy(rows.at[0],
                        agg_sh.at[pl.ds(sid * RPT + k * CS, CS), :])
    plsc.subcore_barrier()

    def group(g, carry):
        for b in range(NBUF):
            @pl.when(g > 0)
            def _():
                pltpu.make_async_copy(
                    rows.at[b], agg_sh.at[idx2.at[b]], sem_c.at[b]).wait()
        da, db = [], []
        for b in range(NBUF):
            j = g * NBUF + b
            off = base + j * CS
            da.append(pltpu.async_copy(
                dst_hbm.at[pl.ds(off, CS)], idx2.at[b], sem_a.at[b]))
            db.append(pltpu.async_copy(
                enew_hbm.at[pl.ds(off, CS), :], rows.at[b], sem_b.at[b]))
        for b in range(NBUF):
            da[b].wait()
            db[b].wait()
            pltpu.async_copy(rows.at[b], agg_sh.at[idx2.at[b]], sem_c.at[b],
                             add=True)
        return carry

    lax.fori_loop(0, NGRPS, group, 0)
    for b in range(NBUF):
        pltpu.make_async_copy(
            rows.at[b], agg_sh.at[idx2.at[b]], sem_c.at[b]).wait()
    plsc.subcore_barrier()

    pltpu.sync_copy(agg_sh.at[pl.ds(sid * RPT, RPT), :],
                    out_hbm.at[cid, pl.ds(sid * RPT, RPT), :])


def _enc_w(enc):
    m = enc["mlp"]
    return (m[0]["w"], m[0]["b"].reshape(1, D),
            m[1]["w"], m[1]["b"].reshape(1, D),
            m[2]["w"], m[2]["b"].reshape(1, D),
            enc["ln_g"].reshape(1, D), enc["ln_b"].reshape(1, D))


def kernel(velocity, node_type, edge_index, edge_features, params):
    src = edge_index[0]
    dst = edge_index[1]
    steps = params["steps"]

    ne_w1, ne_b1, ne_w2, ne_b2, ne_w3, ne_b3, ne_g, ne_b = _enc_w(params["node_enc"])
    wa0 = steps[0]["edge_fn"]["mlp"][0]["w"][0:D]
    wb0 = steps[0]["edge_fn"]["mlp"][0]["w"][D:2 * D]

    x, u, v = pl.pallas_call(
        _node_encode_body,
        grid=(N // BN,),
        in_specs=[_row_spec(BN, 3), _row_spec(BN, 1)] + _wspecs(
            [(3, D), (NTYPES, D), (1, D), (D, D), (1, D), (D, D), (1, D),
             (1, D), (1, D), (D, D), (D, D)]),
        out_specs=[_row_spec(BN, D)] * 3,
        out_shape=[jax.ShapeDtypeStruct((N, D), jnp.float32)] * 3,
    )(velocity, node_type.astype(jnp.int32), ne_w1[0:3], ne_w1[3:12], ne_b1,
      ne_w2, ne_b2, ne_w3, ne_b3, ne_g, ne_b, wa0, wb0)

    ee_w1, ee_b1, ee_w2, ee_b2, ee_w3, ee_b3, ee_g, ee_b = _enc_w(params["edge_enc"])
    srcs = [src[sl * E2:(sl + 1) * E2] for sl in range(NSLAB)]
    dsts = [dst[sl * E2:(sl + 1) * E2] for sl in range(NSLAB)]
    es = [pl.pallas_call(
        _edge_encode_body,
        grid=(E2 // BE,),
        in_specs=[_row_spec(BE, 3)] + _wspecs(
            [(3, D), (1, D), (D, D), (1, D), (D, D), (1, D), (1, D), (1, D)]),
        out_specs=_row_spec(BE, D),
        out_shape=jax.ShapeDtypeStruct((E2, D), jnp.float32),
    )(edge_features[sl * E2:(sl + 1) * E2], ee_w1, ee_b1, ee_w2, ee_b2,
      ee_w3, ee_b3, ee_g, ee_b) for sl in range(NSLAB)]

    for s in range(NSTEPS):
        ef_w1, ef_b1, ef_w2, ef_b2, ef_w3, ef_b3, ef_g, ef_b = _enc_w(steps[s]["edge_fn"])
        aggs = []
        for sl in range(NSLAB):
            g = _gather_g(u, v, srcs[sl], dsts[sl])
            es[sl], enew = pl.pallas_call(
                _edge_step_body,
                grid=(E2 // BE,),
                in_specs=[_row_spec(BE, D), _row_spec(BE, D)] + _wspecs(
                    [(D, D), (1, D), (D, D), (1, D), (D, D), (1, D), (1, D), (1, D)]),
                out_specs=[_row_spec(BE, D)] * 2,
                out_shape=[jax.ShapeDtypeStruct((E2, D), jnp.float32)] * 2,
            )(g, es[sl], ef_w1[2 * D:3 * D], ef_b1, ef_w2, ef_b2, ef_w3,
              ef_b3, ef_g, ef_b)
            aggs.append(_scatter_agg(enew, dsts[sl])[:, :N, :])

        nf_w1, nf_b1, nf_w2, nf_b2, nf_w3, nf_b3, nf_g, nf_b = _enc_w(steps[s]["node_fn"])
        agg_spec = pl.BlockSpec((NC, BN, D), lambda i: (0, i, 0))
        if s < NSTEPS - 1:
            wa = steps[s + 1]["edge_fn"]["mlp"][0]["w"][0:D]
            wb = steps[s + 1]["edge_fn"]["mlp"][0]["w"][D:2 * D]
            x, u, v = pl.pallas_call(
                _node_step_body,
                grid=(N // BN,),
                in_specs=[_row_spec(BN, D)] + [agg_spec] * NSLAB + _wspecs(
                    [(D, D), (D, D), (1, D), (D, D), (1, D), (D, D), (1, D),
                     (1, D), (1, D), (D, D), (D, D)]),
                out_specs=[_row_spec(BN, D)] * 3,
                out_shape=[jax.ShapeDtypeStruct((N, D), jnp.float32)] * 3,
            )(x, *aggs, nf_w1[0:D], nf_w1[D:2 * D], nf_b1, nf_w2,
              nf_b2, nf_w3, nf_b3, nf_g, nf_b, wa, wb)
        else:
            dm = params["decoder"]["mlp"]
            out = pl.pallas_call(
                _final_body,
                grid=(N // BN,),
                in_specs=[_row_spec(BN, D)] + [agg_spec] * NSLAB + _wspecs(
                    [(D, D), (D, D), (1, D), (D, D), (1, D), (D, D), (1, D),
                     (1, D), (1, D),
                     (D, D), (1, D), (D, D), (1, D), (D, 3), (1, 3)]),
                out_specs=_row_spec(BN, 3),
                out_shape=jax.ShapeDtypeStruct((N, 3), jnp.float32),
            )(x, *aggs, nf_w1[0:D], nf_w1[D:2 * D], nf_b1, nf_w2,
              nf_b2, nf_w3, nf_b3, nf_g, nf_b,
              dm[0]["w"], dm[0]["b"].reshape(1, D),
              dm[1]["w"], dm[1]["b"].reshape(1, D),
              dm[2]["w"], dm[2]["b"].reshape(1, 3))
    return out


# speedup vs baseline: 1.0622x; 1.0286x over previous
import functools

import jax
import jax.numpy as jnp
from jax import lax
from jax.experimental import pallas as pl
from jax.experimental.pallas import tpu as pltpu
from jax.experimental.pallas import tpu_sc as plsc

N = 10000
E = 640000
D = 128
NTYPES = 9
NSTEPS = 10
EPS = 1e-5

NC = 2
NS = 16
NW = NC * NS
NSLAB = 2
E2 = E // NSLAB
EPW = E2 // NW
CHUNK = 80
NCH = EPW // CHUNK
NPAD = 10240
RPT = NPAD // NS

BN = 1000
BE = 4000


def _ln(h, g, b):
    m = jnp.mean(h, axis=1, keepdims=True)
    v = jnp.mean((h - m) * (h - m), axis=1, keepdims=True)
    return (h - m) * lax.rsqrt(v + EPS) * g + b


def _mm(a, b):
    return jnp.dot(a, b, preferred_element_type=jnp.float32)


def _node_encode_body(vel, nt, w1v, w1o, b1, w2, b2, w3, b3, lg, lb, wa, wb,
                      x_o, u_o, v_o):
    oh = (nt[...] == lax.broadcasted_iota(jnp.int32, (BN, NTYPES), 1)
          ).astype(jnp.float32)
    h = jnp.maximum(_mm(vel[...], w1v[...]) + _mm(oh, w1o[...]) + b1[...], 0.0)
    h = jnp.maximum(_mm(h, w2[...]) + b2[...], 0.0)
    h = _mm(h, w3[...]) + b3[...]
    x = _ln(h, lg[...], lb[...])
    x_o[...] = x
    u_o[...] = _mm(x, wa[...])
    v_o[...] = _mm(x, wb[...])


def _edge_encode_body(ef, w1, b1, w2, b2, w3, b3, lg, lb, e_o):
    h = jnp.maximum(_mm(ef[...], w1[...]) + b1[...], 0.0)
    h = jnp.maximum(_mm(h, w2[...]) + b2[...], 0.0)
    h = _mm(h, w3[...]) + b3[...]
    e_o[...] = _ln(h, lg[...], lb[...])


def _edge_step_body(g, e, w1c, b1, w2, b2, w3, b3, lg, lb, e_o, enew_o):
    e_in = e[...]
    h = jnp.maximum(g[...] + _mm(e_in, w1c[...]) + b1[...], 0.0)
    h = jnp.maximum(_mm(h, w2[...]) + b2[...], 0.0)
    h = _mm(h, w3[...]) + b3[...]
    enew = _ln(h, lg[...], lb[...])
    enew_o[...] = enew
    e_o[...] = e_in + enew


def _sum_aggs(aggs):
    acc = aggs[0][0] + aggs[0][1]
    for a in aggs[1:]:
        acc = acc + (a[0] + a[1])
    return acc


def _node_step_body(*refs):
    x, aggs, rest = refs[0], refs[1:1 + NSLAB], refs[1 + NSLAB:]
    w1x, w1a, b1, w2, b2, w3, b3, lg, lb, wa, wb, x_o, u_o, v_o = rest
    x_in = x[...]
    agg = _sum_aggs(aggs)
    h = jnp.maximum(_mm(x_in, w1x[...]) + _mm(agg, w1a[...]) + b1[...], 0.0)
    h = jnp.maximum(_mm(h, w2[...]) + b2[...], 0.0)
    h = _mm(h, w3[...]) + b3[...]
    xn = x_in + _ln(h, lg[...], lb[...])
    x_o[...] = xn
    u_o[...] = _mm(xn, wa[...])
    v_o[...] = _mm(xn, wb[...])


def _final_body(*refs):
    x, aggs, rest = refs[0], refs[1:1 + NSLAB], refs[1 + NSLAB:]
    w1x, w1a, b1, w2, b2, w3, b3, lg, lb, dw1, db1, dw2, db2, dw3, db3, out_o = rest
    x_in = x[...]
    agg = _sum_aggs(aggs)
    h = jnp.maximum(_mm(x_in, w1x[...]) + _mm(agg, w1a[...]) + b1[...], 0.0)
    h = jnp.maximum(_mm(h, w2[...]) + b2[...], 0.0)
    h = _mm(h, w3[...]) + b3[...]
    xn = x_in + _ln(h, lg[...], lb[...])
    h = jnp.maximum(_mm(xn, dw1[...]) + db1[...], 0.0)
    h = jnp.maximum(_mm(h, dw2[...]) + db2[...], 0.0)
    out_o[...] = _mm(h, dw3[...]) + db3[...]


def _row_spec(bn, width):
    return pl.BlockSpec((bn, width), lambda i: (i, 0))


def _full_spec(shape):
    nd = len(shape)
    return pl.BlockSpec(shape, lambda i: (0,) * nd)


def _wspecs(shapes):
    return [_full_spec(s) for s in shapes]


_SC_MESH = plsc.VectorSubcoreMesh(core_axis_name="c", subcore_axis_name="s",
                                  num_cores=NC, num_subcores=NS)

NBUF = 5
NGRP = NCH // NBUF


@functools.partial(
    pl.kernel,
    out_type=jax.ShapeDtypeStruct((E2, D), jnp.float32),
    mesh=_SC_MESH,
    scratch_types=[
        pltpu.VMEM((EPW,), jnp.int32),
        pltpu.VMEM((EPW,), jnp.int32),
        pltpu.VMEM((NBUF, CHUNK, D), jnp.float32),
        pltpu.SemaphoreType.DMA((NBUF,)),
        pltpu.SemaphoreType.DMA((NBUF,)),
        pltpu.SemaphoreType.DMA((NBUF,)),
    ],
)
def _gather_g(u_hbm, v_hbm, src_hbm, dst_hbm, g_hbm, sidx, didx, rows,
              sem_u, sem_v, sem_d):
    wid = lax.axis_index("s") * NC + lax.axis_index("c")
    base = wid * EPW
    pltpu.sync_copy(src_hbm.at[pl.ds(base, EPW)], sidx)
    pltpu.sync_copy(dst_hbm.at[pl.ds(base, EPW)], didx)

    def group(g, carry):
        for b in range(NBUF):
            @pl.when(g > 0)
            def _():
                pltpu.make_async_copy(
                    rows.at[b], g_hbm.at[pl.ds(base, CHUNK), :], sem_d.at[b]
                ).wait()
        du = []
        for b in range(NBUF):
            j = g * NBUF + b
            du.append(pltpu.async_copy(
                u_hbm.at[sidx.at[pl.ds(j * CHUNK, CHUNK)]], rows.at[b],
                sem_u.at[b]))
        dv = []
        for b in range(NBUF):
            j = g * NBUF + b
            du[b].wait()
            dv.append(pltpu.async_copy(
                v_hbm.at[didx.at[pl.ds(j * CHUNK, CHUNK)]], rows.at[b],
                sem_v.at[b], add=True))
        for b in range(NBUF):
            j = g * NBUF + b
            dv[b].wait()
            pltpu.async_copy(rows.at[b],
                             g_hbm.at[pl.ds(base + j * CHUNK, CHUNK), :],
                             sem_d.at[b])
        return carry

    lax.fori_loop(0, NGRP, group, 0)
    for b in range(NBUF):
        pltpu.make_async_copy(
            rows.at[b], g_hbm.at[pl.ds(base, CHUNK), :], sem_d.at[b]).wait()


CS = 40
NCHS = EPW // CS
NGRPS = NCHS // NBUF


@functools.partial(
    pl.kernel,
    out_type=jax.ShapeDtypeStruct((NC, NPAD, D), jnp.float32),
    mesh=_SC_MESH,
    scratch_types=[
        pltpu.VMEM((NBUF, CS), jnp.int32),
        pltpu.VMEM((NBUF, CS, D), jnp.float32),
        pltpu.VMEM_SHARED((NPAD, D), jnp.float32),
        pltpu.SemaphoreType.DMA((NBUF,)),
        pltpu.SemaphoreType.DMA((NBUF,)),
        pltpu.SemaphoreType.DMA((NBUF,)),
    ],
)
def _scatter_agg(enew_hbm, dst_hbm, out_hbm, idx2, rows, agg_sh,
                 sem_a, sem_b, sem_c):
    cid = lax.axis_index("c")
    sid = lax.axis_index("s")
    wid = sid * NC + cid
    base = wid * EPW

    def zr(r, carry):
        def zl(l, c2):
            rows[0, r, pl.ds(l * 16, 16)] = jnp.zeros((16,), jnp.float32)
            return c2
        return lax.fori_loop(0, 8, zl, carry)

    lax.fori_loop(0, CS, zr, 0)
    for k in range(RPT // CS):
        pltpu.sync_copy(rows.at[0],
                        agg_sh.at[pl.ds(sid * RPT + k * CS, CS), :])
    plsc.subcore_barrier()

    def group(g, carry):
        for b in range(NBUF):
            @pl.when(g > 0)
            def _():
                pltpu.make_async_copy(
                    rows.at[b], agg_sh.at[idx2.at[b]], sem_c.at[b]).wait()
        da, db = [], []
        for b in range(NBUF):
            j = g * NBUF + b
            off = base + j * CS
            da.append(pltpu.async_copy(
                dst_hbm.at[pl.ds(off, CS)], idx2.at[b], sem_a.at[b]))
            db.append(pltpu.async_copy(
                enew_hbm.at[pl.ds(off, CS), :], rows.at[b], sem_b.at[b]))
        for b in range(NBUF):
            da[b].wait()
            db[b].wait()
            pltpu.async_copy(rows.at[b], agg_sh.at[idx2.at[b]], sem_c.at[b],
                             add=True)
        return carry

    lax.fori_loop(0, NGRPS, group, 0)
    for b in range(NBUF):
        pltpu.make_async_copy(
            rows.at[b], agg_sh.at[idx2.at[b]], sem_c.at[b]).wait()
    plsc.subcore_barrier()

    pltpu.sync_copy(agg_sh.at[pl.ds(sid * RPT, RPT), :],
                    out_hbm.at[cid, pl.ds(sid * RPT, RPT), :])


def _enc_w(enc):
    m = enc["mlp"]
    return (m[0]["w"], m[0]["b"].reshape(1, D),
            m[1]["w"], m[1]["b"].reshape(1, D),
            m[2]["w"], m[2]["b"].reshape(1, D),
            enc["ln_g"].reshape(1, D), enc["ln_b"].reshape(1, D))


def kernel(velocity, node_type, edge_index, edge_features, params):
    src = edge_index[0]
    dst = edge_index[1]
    steps = params["steps"]

    ne_w1, ne_b1, ne_w2, ne_b2, ne_w3, ne_b3, ne_g, ne_b = _enc_w(params["node_enc"])
    wa0 = steps[0]["edge_fn"]["mlp"][0]["w"][0:D]
    wb0 = steps[0]["edge_fn"]["mlp"][0]["w"][D:2 * D]

    x, u, v = pl.pallas_call(
        _node_encode_body,
        grid=(N // BN,),
        in_specs=[_row_spec(BN, 3), _row_spec(BN, 1)] + _wspecs(
            [(3, D), (NTYPES, D), (1, D), (D, D), (1, D), (D, D), (1, D),
             (1, D), (1, D), (D, D), (D, D)]),
        out_specs=[_row_spec(BN, D)] * 3,
        out_shape=[jax.ShapeDtypeStruct((N, D), jnp.float32)] * 3,
    )(velocity, node_type.astype(jnp.int32), ne_w1[0:3], ne_w1[3:12], ne_b1,
      ne_w2, ne_b2, ne_w3, ne_b3, ne_g, ne_b, wa0, wb0)

    ee_w1, ee_b1, ee_w2, ee_b2, ee_w3, ee_b3, ee_g, ee_b = _enc_w(params["edge_enc"])
    srcs = [src[sl * E2:(sl + 1) * E2] for sl in range(NSLAB)]
    dsts = [dst[sl * E2:(sl + 1) * E2] for sl in range(NSLAB)]
    es = [pl.pallas_call(
        _edge_encode_body,
        grid=(E2 // BE,),
        in_specs=[_row_spec(BE, 3)] + _wspecs(
            [(3, D), (1, D), (D, D), (1, D), (D, D), (1, D), (1, D), (1, D)]),
        out_specs=_row_spec(BE, D),
        out_shape=jax.ShapeDtypeStruct((E2, D), jnp.float32),
    )(edge_features[sl * E2:(sl + 1) * E2], ee_w1, ee_b1, ee_w2, ee_b2,
      ee_w3, ee_b3, ee_g, ee_b) for sl in range(NSLAB)]

    for s in range(NSTEPS):
        ef_w1, ef_b1, ef_w2, ef_b2, ef_w3, ef_b3, ef_g, ef_b = _enc_w(steps[s]["edge_fn"])
        aggs = []
        for sl in range(NSLAB):
            g = _gather_g(u, v, srcs[sl], dsts[sl])
            es[sl], enew = pl.pallas_call(
                _edge_step_body,
                grid=(E2 // BE,),
                in_specs=[_row_spec(BE, D), _row_spec(BE, D)] + _wspecs(
                    [(D, D), (1, D), (D, D), (1, D), (D, D), (1, D), (1, D), (1, D)]),
                out_specs=[_row_spec(BE, D)] * 2,
                out_shape=[jax.ShapeDtypeStruct((E2, D), jnp.float32)] * 2,
            )(g, es[sl], ef_w1[2 * D:3 * D], ef_b1, ef_w2, ef_b2, ef_w3,
              ef_b3, ef_g, ef_b)
            aggs.append(_scatter_agg(enew, dsts[sl])[:, :N, :])

        nf_w1, nf_b1, nf_w2, nf_b2, nf_w3, nf_b3, nf_g, nf_b = _enc_w(steps[s]["node_fn"])
        agg_spec = pl.BlockSpec((NC, BN, D), lambda i: (0, i, 0))
        if s < NSTEPS - 1:
            wa = steps[s + 1]["edge_fn"]["mlp"][0]["w"][0:D]
            wb = steps[s + 1]["edge_fn"]["mlp"][0]["w"][D:2 * D]
            x, u, v = pl.pallas_call(
                _node_step_body,
                grid=(N // BN,),
                in_specs=[_row_spec(BN, D)] + [agg_spec] * NSLAB + _wspecs(
                    [(D, D), (D, D), (1, D), (D, D), (1, D), (D, D), (1, D),
                     (1, D), (1, D), (D, D), (D, D)]),
                out_specs=[_row_spec(BN, D)] * 3,
                out_shape=[jax.ShapeDtypeStruct((N, D), jnp.float32)] * 3,
            )(x, *aggs, nf_w1[0:D], nf_w1[D:2 * D], nf_b1, nf_w2,
              nf_b2, nf_w3, nf_b3, nf_g, nf_b, wa, wb)
        else:
            dm = params["decoder"]["mlp"]
            out = pl.pallas_call(
                _final_body,
                grid=(N // BN,),
                in_specs=[_row_spec(BN, D)] + [agg_spec] * NSLAB + _wspecs(
                    [(D, D), (D, D), (1, D), (D, D), (1, D), (D, D), (1, D),
                     (1, D), (1, D),
                     (D, D), (1, D), (D, D), (1, D), (D, 3), (1, 3)]),
                out_specs=_row_spec(BN, 3),
                out_shape=jax.ShapeDtypeStruct((N, 3), jnp.float32),
            )(x, *aggs, nf_w1[0:D], nf_w1[D:2 * D], nf_b1, nf_w2,
              nf_b2, nf_w3, nf_b3, nf_g, nf_b,
              dm[0]["w"], dm[0]["b"].reshape(1, D),
              dm[1]["w"], dm[1]["b"].reshape(1, D),
              dm[2]["w"], dm[2]["b"].reshape(1, 3))
    return out


# speedup vs baseline: 1.0718x; 1.0090x over previous
import functools

import jax
import jax.numpy as jnp
from jax import lax
from jax.experimental import pallas as pl
from jax.experimental.pallas import tpu as pltpu
from jax.experimental.pallas import tpu_sc as plsc

N = 10000
E = 640000
D = 128
NTYPES = 9
NSTEPS = 10
EPS = 1e-5

NC = 2
NS = 16
NW = NC * NS
NSLAB = 2
E2 = E // NSLAB
EPW = E2 // NW
CHUNK = 80
NCH = EPW // CHUNK
NPAD = 10240
RPT = NPAD // NS

BN = 1000
BE = 8000


def _ln(h, g, b):
    m = jnp.mean(h, axis=1, keepdims=True)
    v = jnp.mean((h - m) * (h - m), axis=1, keepdims=True)
    return (h - m) * lax.rsqrt(v + EPS) * g + b


def _mm(a, b):
    return jnp.dot(a, b, preferred_element_type=jnp.float32)


def _node_encode_body(vel, nt, w1v, w1o, b1, w2, b2, w3, b3, lg, lb, wa, wb,
                      x_o, u_o, v_o):
    oh = (nt[...] == lax.broadcasted_iota(jnp.int32, (BN, NTYPES), 1)
          ).astype(jnp.float32)
    h = jnp.maximum(_mm(vel[...], w1v[...]) + _mm(oh, w1o[...]) + b1[...], 0.0)
    h = jnp.maximum(_mm(h, w2[...]) + b2[...], 0.0)
    h = _mm(h, w3[...]) + b3[...]
    x = _ln(h, lg[...], lb[...])
    x_o[...] = x
    u_o[...] = _mm(x, wa[...])
    v_o[...] = _mm(x, wb[...])


def _edge_encode_body(ef, w1, b1, w2, b2, w3, b3, lg, lb, e_o):
    h = jnp.maximum(_mm(ef[...], w1[...]) + b1[...], 0.0)
    h = jnp.maximum(_mm(h, w2[...]) + b2[...], 0.0)
    h = _mm(h, w3[...]) + b3[...]
    e_o[...] = _ln(h, lg[...], lb[...])


def _edge_step_body(g, e, w1c, b1, w2, b2, w3, b3, lg, lb, e_o, enew_o):
    e_in = e[...]
    h = jnp.maximum(g[...] + _mm(e_in, w1c[...]) + b1[...], 0.0)
    h = jnp.maximum(_mm(h, w2[...]) + b2[...], 0.0)
    h = _mm(h, w3[...]) + b3[...]
    enew = _ln(h, lg[...], lb[...])
    enew_o[...] = enew
    e_o[...] = e_in + enew


def _sum_aggs(aggs):
    acc = aggs[0][0] + aggs[0][1]
    for a in aggs[1:]:
        acc = acc + (a[0] + a[1])
    return acc


def _node_step_body(*refs):
    x, aggs, rest = refs[0], refs[1:1 + NSLAB], refs[1 + NSLAB:]
    w1x, w1a, b1, w2, b2, w3, b3, lg, lb, wa, wb, x_o, u_o, v_o = rest
    x_in = x[...]
    agg = _sum_aggs(aggs)
    h = jnp.maximum(_mm(x_in, w1x[...]) + _mm(agg, w1a[...]) + b1[...], 0.0)
    h = jnp.maximum(_mm(h, w2[...]) + b2[...], 0.0)
    h = _mm(h, w3[...]) + b3[...]
    xn = x_in + _ln(h, lg[...], lb[...])
    x_o[...] = xn
    u_o[...] = _mm(xn, wa[...])
    v_o[...] = _mm(xn, wb[...])


def _final_body(*refs):
    x, aggs, rest = refs[0], refs[1:1 + NSLAB], refs[1 + NSLAB:]
    w1x, w1a, b1, w2, b2, w3, b3, lg, lb, dw1, db1, dw2, db2, dw3, db3, out_o = rest
    x_in = x[...]
    agg = _sum_aggs(aggs)
    h = jnp.maximum(_mm(x_in, w1x[...]) + _mm(agg, w1a[...]) + b1[...], 0.0)
    h = jnp.maximum(_mm(h, w2[...]) + b2[...], 0.0)
    h = _mm(h, w3[...]) + b3[...]
    xn = x_in + _ln(h, lg[...], lb[...])
    h = jnp.maximum(_mm(xn, dw1[...]) + db1[...], 0.0)
    h = jnp.maximum(_mm(h, dw2[...]) + db2[...], 0.0)
    out_o[...] = _mm(h, dw3[...]) + db3[...]


def _row_spec(bn, width):
    return pl.BlockSpec((bn, width), lambda i: (i, 0))


def _full_spec(shape):
    nd = len(shape)
    return pl.BlockSpec(shape, lambda i: (0,) * nd)


def _wspecs(shapes):
    return [_full_spec(s) for s in shapes]


_SC_MESH = plsc.VectorSubcoreMesh(core_axis_name="c", subcore_axis_name="s",
                                  num_cores=NC, num_subcores=NS)

NBUF = 5
NGRP = NCH // NBUF


@functools.partial(
    pl.kernel,
    out_type=jax.ShapeDtypeStruct((E2, D), jnp.float32),
    mesh=_SC_MESH,
    scratch_types=[
        pltpu.VMEM((EPW,), jnp.int32),
        pltpu.VMEM((EPW,), jnp.int32),
        pltpu.VMEM((NBUF, CHUNK, D), jnp.float32),
        pltpu.SemaphoreType.DMA((NBUF,)),
        pltpu.SemaphoreType.DMA((NBUF,)),
        pltpu.SemaphoreType.DMA((NBUF,)),
    ],
)
def _gather_g(u_hbm, v_hbm, src_hbm, dst_hbm, g_hbm, sidx, didx, rows,
              sem_u, sem_v, sem_d):
    wid = lax.axis_index("s") * NC + lax.axis_index("c")
    base = wid * EPW
    pltpu.sync_copy(src_hbm.at[pl.ds(base, EPW)], sidx)
    pltpu.sync_copy(dst_hbm.at[pl.ds(base, EPW)], didx)

    def group(g, carry):
        for b in range(NBUF):
            @pl.when(g > 0)
            def _():
                pltpu.make_async_copy(
                    rows.at[b], g_hbm.at[pl.ds(base, CHUNK), :], sem_d.at[b]
                ).wait()
        du = []
        for b in range(NBUF):
            j = g * NBUF + b
            du.append(pltpu.async_copy(
                u_hbm.at[sidx.at[pl.ds(j * CHUNK, CHUNK)]], rows.at[b],
                sem_u.at[b]))
        dv = []
        for b in range(NBUF):
            j = g * NBUF + b
            du[b].wait()
            dv.append(pltpu.async_copy(
                v_hbm.at[didx.at[pl.ds(j * CHUNK, CHUNK)]], rows.at[b],
                sem_v.at[b], add=True))
        for b in range(NBUF):
            j = g * NBUF + b
            dv[b].wait()
            pltpu.async_copy(rows.at[b],
                             g_hbm.at[pl.ds(base + j * CHUNK, CHUNK), :],
                             sem_d.at[b])
        return carry

    lax.fori_loop(0, NGRP, group, 0)
    for b in range(NBUF):
        pltpu.make_async_copy(
            rows.at[b], g_hbm.at[pl.ds(base, CHUNK), :], sem_d.at[b]).wait()


CS = 40
NCHS = EPW // CS
NGRPS = NCHS // NBUF


@functools.partial(
    pl.kernel,
    out_type=jax.ShapeDtypeStruct((NC, NPAD, D), jnp.float32),
    mesh=_SC_MESH,
    scratch_types=[
        pltpu.VMEM((NBUF, CS), jnp.int32),
        pltpu.VMEM((NBUF, CS, D), jnp.float32),
        pltpu.VMEM_SHARED((NPAD, D), jnp.float32),
        pltpu.SemaphoreType.DMA((NBUF,)),
        pltpu.SemaphoreType.DMA((NBUF,)),
        pltpu.SemaphoreType.DMA((NBUF,)),
    ],
)
def _scatter_agg(enew_hbm, dst_hbm, out_hbm, idx2, rows, agg_sh,
                 sem_a, sem_b, sem_c):
    cid = lax.axis_index("c")
    sid = lax.axis_index("s")
    wid = sid * NC + cid
    base = wid * EPW

    def zr(r, carry):
        def zl(l, c2):
            rows[0, r, pl.ds(l * 16, 16)] = jnp.zeros((16,), jnp.float32)
            return c2
        return lax.fori_loop(0, 8, zl, carry)

    lax.fori_loop(0, CS, zr, 0)
    for k in range(RPT // CS):
        pltpu.sync_copy(rows.at[0],
                        agg_sh.at[pl.ds(sid * RPT + k * CS, CS), :])
    plsc.subcore_barrier()

    def group(g, carry):
        for b in range(NBUF):
            @pl.when(g > 0)
            def _():
                pltpu.make_async_copy(
                    rows.at[b], agg_sh.at[idx2.at[b]], sem_c.at[b]).wait()
        da, db = [], []
        for b in range(NBUF):
            j = g * NBUF + b
            off = base + j * CS
            da.append(pltpu.async_copy(
                dst_hbm.at[pl.ds(off, CS)], idx2.at[b], sem_a.at[b]))
            db.append(pltpu.async_copy(
                enew_hbm.at[pl.ds(off, CS), :], rows.at[b], sem_b.at[b]))
        for b in range(NBUF):
            da[b].wait()
            db[b].wait()
            pltpu.async_copy(rows.at[b], agg_sh.at[idx2.at[b]], sem_c.at[b],
                             add=True)
        return carry

    lax.fori_loop(0, NGRPS, group, 0)
    for b in range(NBUF):
        pltpu.make_async_copy(
            rows.at[b], agg_sh.at[idx2.at[b]], sem_c.at[b]).wait()
    plsc.subcore_barrier()

    pltpu.sync_copy(agg_sh.at[pl.ds(sid * RPT, RPT), :],
                    out_hbm.at[cid, pl.ds(sid * RPT, RPT), :])


def _enc_w(enc):
    m = enc["mlp"]
    return (m[0]["w"], m[0]["b"].reshape(1, D),
            m[1]["w"], m[1]["b"].reshape(1, D),
            m[2]["w"], m[2]["b"].reshape(1, D),
            enc["ln_g"].reshape(1, D), enc["ln_b"].reshape(1, D))


def kernel(velocity, node_type, edge_index, edge_features, params):
    src = edge_index[0]
    dst = edge_index[1]
    steps = params["steps"]

    ne_w1, ne_b1, ne_w2, ne_b2, ne_w3, ne_b3, ne_g, ne_b = _enc_w(params["node_enc"])
    wa0 = steps[0]["edge_fn"]["mlp"][0]["w"][0:D]
    wb0 = steps[0]["edge_fn"]["mlp"][0]["w"][D:2 * D]

    x, u, v = pl.pallas_call(
        _node_encode_body,
        grid=(N // BN,),
        in_specs=[_row_spec(BN, 3), _row_spec(BN, 1)] + _wspecs(
            [(3, D), (NTYPES, D), (1, D), (D, D), (1, D), (D, D), (1, D),
             (1, D), (1, D), (D, D), (D, D)]),
        out_specs=[_row_spec(BN, D)] * 3,
        out_shape=[jax.ShapeDtypeStruct((N, D), jnp.float32)] * 3,
    )(velocity, node_type.astype(jnp.int32), ne_w1[0:3], ne_w1[3:12], ne_b1,
      ne_w2, ne_b2, ne_w3, ne_b3, ne_g, ne_b, wa0, wb0)

    ee_w1, ee_b1, ee_w2, ee_b2, ee_w3, ee_b3, ee_g, ee_b = _enc_w(params["edge_enc"])
    srcs = [src[sl * E2:(sl + 1) * E2] for sl in range(NSLAB)]
    dsts = [dst[sl * E2:(sl + 1) * E2] for sl in range(NSLAB)]
    es = [pl.pallas_call(
        _edge_encode_body,
        grid=(E2 // BE,),
        in_specs=[_row_spec(BE, 3)] + _wspecs(
            [(3, D), (1, D), (D, D), (1, D), (D, D), (1, D), (1, D), (1, D)]),
        out_specs=_row_spec(BE, D),
        out_shape=jax.ShapeDtypeStruct((E2, D), jnp.float32),
    )(edge_features[sl * E2:(sl + 1) * E2], ee_w1, ee_b1, ee_w2, ee_b2,
      ee_w3, ee_b3, ee_g, ee_b) for sl in range(NSLAB)]

    for s in range(NSTEPS):
        ef_w1, ef_b1, ef_w2, ef_b2, ef_w3, ef_b3, ef_g, ef_b = _enc_w(steps[s]["edge_fn"])
        aggs = []
        for sl in range(NSLAB):
            g = _gather_g(u, v, srcs[sl], dsts[sl])
            es[sl], enew = pl.pallas_call(
                _edge_step_body,
                grid=(E2 // BE,),
                in_specs=[_row_spec(BE, D), _row_spec(BE, D)] + _wspecs(
                    [(D, D), (1, D), (D, D), (1, D), (D, D), (1, D), (1, D), (1, D)]),
                out_specs=[_row_spec(BE, D)] * 2,
                out_shape=[jax.ShapeDtypeStruct((E2, D), jnp.float32)] * 2,
            )(g, es[sl], ef_w1[2 * D:3 * D], ef_b1, ef_w2, ef_b2, ef_w3,
              ef_b3, ef_g, ef_b)
            aggs.append(_scatter_agg(enew, dsts[sl])[:, :N, :])

        nf_w1, nf_b1, nf_w2, nf_b2, nf_w3, nf_b3, nf_g, nf_b = _enc_w(steps[s]["node_fn"])
        agg_spec = pl.BlockSpec((NC, BN, D), lambda i: (0, i, 0))
        if s < NSTEPS - 1:
            wa = steps[s + 1]["edge_fn"]["mlp"][0]["w"][0:D]
            wb = steps[s + 1]["edge_fn"]["mlp"][0]["w"][D:2 * D]
            x, u, v = pl.pallas_call(
                _node_step_body,
                grid=(N // BN,),
                in_specs=[_row_spec(BN, D)] + [agg_spec] * NSLAB + _wspecs(
                    [(D, D), (D, D), (1, D), (D, D), (1, D), (D, D), (1, D),
                     (1, D), (1, D), (D, D), (D, D)]),
                out_specs=[_row_spec(BN, D)] * 3,
                out_shape=[jax.ShapeDtypeStruct((N, D), jnp.float32)] * 3,
            )(x, *aggs, nf_w1[0:D], nf_w1[D:2 * D], nf_b1, nf_w2,
              nf_b2, nf_w3, nf_b3, nf_g, nf_b, wa, wb)
        else:
            dm = params["decoder"]["mlp"]
            out = pl.pallas_call(
                _final_body,
                grid=(N // BN,),
                in_specs=[_row_spec(BN, D)] + [agg_spec] * NSLAB + _wspecs(
                    [(D, D), (D, D), (1, D), (D, D), (1, D), (D, D), (1, D),
                     (1, D), (1, D),
                     (D, D), (1, D), (D, D), (1, D), (D, 3), (1, 3)]),
                out_specs=_row_spec(BN, 3),
                out_shape=jax.ShapeDtypeStruct((N, 3), jnp.float32),
            )(x, *aggs, nf_w1[0:D], nf_w1[D:2 * D], nf_b1, nf_w2,
              nf_b2, nf_w3, nf_b3, nf_g, nf_b,
              dm[0]["w"], dm[0]["b"].reshape(1, D),
              dm[1]["w"], dm[1]["b"].reshape(1, D),
              dm[2]["w"], dm[2]["b"].reshape(1, 3))
    return out


# speedup vs baseline: 1.0741x; 1.0022x over previous
import functools

import jax
import jax.numpy as jnp
from jax import lax
from jax.experimental import pallas as pl
from jax.experimental.pallas import tpu as pltpu
from jax.experimental.pallas import tpu_sc as plsc

N = 10000
E = 640000
D = 128
NTYPES = 9
NSTEPS = 10
EPS = 1e-5

NC = 2
NS = 16
NW = NC * NS
NSLAB = 2
E2 = E // NSLAB
EPW = E2 // NW
CHUNK = 80
NCH = EPW // CHUNK
NPAD = 10240
RPT = NPAD // NS

BN = 2000
BE = 8000


def _ln(h, g, b):
    m = jnp.mean(h, axis=1, keepdims=True)
    v = jnp.mean((h - m) * (h - m), axis=1, keepdims=True)
    return (h - m) * lax.rsqrt(v + EPS) * g + b


def _mm(a, b):
    return jnp.dot(a, b, preferred_element_type=jnp.float32)


def _node_encode_body(vel, nt, w1v, w1o, b1, w2, b2, w3, b3, lg, lb, wa, wb,
                      x_o, u_o, v_o):
    oh = (nt[...] == lax.broadcasted_iota(jnp.int32, (BN, NTYPES), 1)
          ).astype(jnp.float32)
    h = jnp.maximum(_mm(vel[...], w1v[...]) + _mm(oh, w1o[...]) + b1[...], 0.0)
    h = jnp.maximum(_mm(h, w2[...]) + b2[...], 0.0)
    h = _mm(h, w3[...]) + b3[...]
    x = _ln(h, lg[...], lb[...])
    x_o[...] = x
    u_o[...] = _mm(x, wa[...])
    v_o[...] = _mm(x, wb[...])


def _edge_encode_body(ef, w1, b1, w2, b2, w3, b3, lg, lb, e_o):
    h = jnp.maximum(_mm(ef[...], w1[...]) + b1[...], 0.0)
    h = jnp.maximum(_mm(h, w2[...]) + b2[...], 0.0)
    h = _mm(h, w3[...]) + b3[...]
    e_o[...] = _ln(h, lg[...], lb[...])


def _edge_step_body(g, e, w1c, b1, w2, b2, w3, b3, lg, lb, e_o, enew_o):
    e_in = e[...]
    h = jnp.maximum(g[...] + _mm(e_in, w1c[...]) + b1[...], 0.0)
    h = jnp.maximum(_mm(h, w2[...]) + b2[...], 0.0)
    h = _mm(h, w3[...]) + b3[...]
    enew = _ln(h, lg[...], lb[...])
    enew_o[...] = enew
    e_o[...] = e_in + enew


def _sum_aggs(aggs):
    acc = aggs[0][0] + aggs[0][1]
    for a in aggs[1:]:
        acc = acc + (a[0] + a[1])
    return acc


def _node_step_body(*refs):
    x, aggs, rest = refs[0], refs[1:1 + NSLAB], refs[1 + NSLAB:]
    w1x, w1a, b1, w2, b2, w3, b3, lg, lb, wa, wb, x_o, u_o, v_o = rest
    x_in = x[...]
    agg = _sum_aggs(aggs)
    h = jnp.maximum(_mm(x_in, w1x[...]) + _mm(agg, w1a[...]) + b1[...], 0.0)
    h = jnp.maximum(_mm(h, w2[...]) + b2[...], 0.0)
    h = _mm(h, w3[...]) + b3[...]
    xn = x_in + _ln(h, lg[...], lb[...])
    x_o[...] = xn
    u_o[...] = _mm(xn, wa[...])
    v_o[...] = _mm(xn, wb[...])


def _final_body(*refs):
    x, aggs, rest = refs[0], refs[1:1 + NSLAB], refs[1 + NSLAB:]
    w1x, w1a, b1, w2, b2, w3, b3, lg, lb, dw1, db1, dw2, db2, dw3, db3, out_o = rest
    x_in = x[...]
    agg = _sum_aggs(aggs)
    h = jnp.maximum(_mm(x_in, w1x[...]) + _mm(agg, w1a[...]) + b1[...], 0.0)
    h = jnp.maximum(_mm(h, w2[...]) + b2[...], 0.0)
    h = _mm(h, w3[...]) + b3[...]
    xn = x_in + _ln(h, lg[...], lb[...])
    h = jnp.maximum(_mm(xn, dw1[...]) + db1[...], 0.0)
    h = jnp.maximum(_mm(h, dw2[...]) + db2[...], 0.0)
    out_o[...] = _mm(h, dw3[...]) + db3[...]


def _row_spec(bn, width):
    return pl.BlockSpec((bn, width), lambda i: (i, 0))


def _full_spec(shape):
    nd = len(shape)
    return pl.BlockSpec(shape, lambda i: (0,) * nd)


def _wspecs(shapes):
    return [_full_spec(s) for s in shapes]


_SC_MESH = plsc.VectorSubcoreMesh(core_axis_name="c", subcore_axis_name="s",
                                  num_cores=NC, num_subcores=NS)

NBUF = 5
NGRP = NCH // NBUF


@functools.partial(
    pl.kernel,
    out_type=jax.ShapeDtypeStruct((E2, D), jnp.float32),
    mesh=_SC_MESH,
    scratch_types=[
        pltpu.VMEM((EPW,), jnp.int32),
        pltpu.VMEM((EPW,), jnp.int32),
        pltpu.VMEM((NBUF, CHUNK, D), jnp.float32),
        pltpu.SemaphoreType.DMA((NBUF,)),
        pltpu.SemaphoreType.DMA((NBUF,)),
        pltpu.SemaphoreType.DMA((NBUF,)),
    ],
)
def _gather_g(u_hbm, v_hbm, src_hbm, dst_hbm, g_hbm, sidx, didx, rows,
              sem_u, sem_v, sem_d):
    wid = lax.axis_index("s") * NC + lax.axis_index("c")
    base = wid * EPW
    pltpu.sync_copy(src_hbm.at[pl.ds(base, EPW)], sidx)
    pltpu.sync_copy(dst_hbm.at[pl.ds(base, EPW)], didx)

    def group(g, carry):
        for b in range(NBUF):
            @pl.when(g > 0)
            def _():
                pltpu.make_async_copy(
                    rows.at[b], g_hbm.at[pl.ds(base, CHUNK), :], sem_d.at[b]
                ).wait()
        du = []
        for b in range(NBUF):
            j = g * NBUF + b
            du.append(pltpu.async_copy(
                u_hbm.at[sidx.at[pl.ds(j * CHUNK, CHUNK)]], rows.at[b],
                sem_u.at[b]))
        dv = []
        for b in range(NBUF):
            j = g * NBUF + b
            du[b].wait()
            dv.append(pltpu.async_copy(
                v_hbm.at[didx.at[pl.ds(j * CHUNK, CHUNK)]], rows.at[b],
                sem_v.at[b], add=True))
        for b in range(NBUF):
            j = g * NBUF + b
            dv[b].wait()
            pltpu.async_copy(rows.at[b],
                             g_hbm.at[pl.ds(base + j * CHUNK, CHUNK), :],
                             sem_d.at[b])
        return carry

    lax.fori_loop(0, NGRP, group, 0)
    for b in range(NBUF):
        pltpu.make_async_copy(
            rows.at[b], g_hbm.at[pl.ds(base, CHUNK), :], sem_d.at[b]).wait()


CS = 40
NCHS = EPW // CS
NGRPS = NCHS // NBUF


@functools.partial(
    pl.kernel,
    out_type=jax.ShapeDtypeStruct((NC, NPAD, D), jnp.float32),
    mesh=_SC_MESH,
    scratch_types=[
        pltpu.VMEM((NBUF, CS), jnp.int32),
        pltpu.VMEM((NBUF, CS, D), jnp.float32),
        pltpu.VMEM_SHARED((NPAD, D), jnp.float32),
        pltpu.SemaphoreType.DMA((NBUF,)),
        pltpu.SemaphoreType.DMA((NBUF,)),
        pltpu.SemaphoreType.DMA((NBUF,)),
    ],
)
def _scatter_agg(enew_hbm, dst_hbm, out_hbm, idx2, rows, agg_sh,
                 sem_a, sem_b, sem_c):
    cid = lax.axis_index("c")
    sid = lax.axis_index("s")
    wid = sid * NC + cid
    base = wid * EPW

    def zr(r, carry):
        def zl(l, c2):
            rows[0, r, pl.ds(l * 16, 16)] = jnp.zeros((16,), jnp.float32)
            return c2
        return lax.fori_loop(0, 8, zl, carry)

    lax.fori_loop(0, CS, zr, 0)
    for k in range(RPT // CS):
        pltpu.sync_copy(rows.at[0],
                        agg_sh.at[pl.ds(sid * RPT + k * CS, CS), :])
    plsc.subcore_barrier()

    def group(g, carry):
        for b in range(NBUF):
            @pl.when(g > 0)
            def _():
                pltpu.make_async_copy(
                    rows.at[b], agg_sh.at[idx2.at[b]], sem_c.at[b]).wait()
        da, db = [], []
        for b in range(NBUF):
            j = g * NBUF + b
            off = base + j * CS
            da.append(pltpu.async_copy(
                dst_hbm.at[pl.ds(off, CS)], idx2.at[b], sem_a.at[b]))
            db.append(pltpu.async_copy(
                enew_hbm.at[pl.ds(off, CS), :], rows.at[b], sem_b.at[b]))
        for b in range(NBUF):
            da[b].wait()
            db[b].wait()
            pltpu.async_copy(rows.at[b], agg_sh.at[idx2.at[b]], sem_c.at[b],
                             add=True)
        return carry

    lax.fori_loop(0, NGRPS, group, 0)
    for b in range(NBUF):
        pltpu.make_async_copy(
            rows.at[b], agg_sh.at[idx2.at[b]], sem_c.at[b]).wait()
    plsc.subcore_barrier()

    pltpu.sync_copy(agg_sh.at[pl.ds(sid * RPT, RPT), :],
                    out_hbm.at[cid, pl.ds(sid * RPT, RPT), :])


def _enc_w(enc):
    m = enc["mlp"]
    return (m[0]["w"], m[0]["b"].reshape(1, D),
            m[1]["w"], m[1]["b"].reshape(1, D),
            m[2]["w"], m[2]["b"].reshape(1, D),
            enc["ln_g"].reshape(1, D), enc["ln_b"].reshape(1, D))


def kernel(velocity, node_type, edge_index, edge_features, params):
    src = edge_index[0]
    dst = edge_index[1]
    steps = params["steps"]

    ne_w1, ne_b1, ne_w2, ne_b2, ne_w3, ne_b3, ne_g, ne_b = _enc_w(params["node_enc"])
    wa0 = steps[0]["edge_fn"]["mlp"][0]["w"][0:D]
    wb0 = steps[0]["edge_fn"]["mlp"][0]["w"][D:2 * D]

    x, u, v = pl.pallas_call(
        _node_encode_body,
        grid=(N // BN,),
        in_specs=[_row_spec(BN, 3), _row_spec(BN, 1)] + _wspecs(
            [(3, D), (NTYPES, D), (1, D), (D, D), (1, D), (D, D), (1, D),
             (1, D), (1, D), (D, D), (D, D)]),
        out_specs=[_row_spec(BN, D)] * 3,
        out_shape=[jax.ShapeDtypeStruct((N, D), jnp.float32)] * 3,
    )(velocity, node_type.astype(jnp.int32), ne_w1[0:3], ne_w1[3:12], ne_b1,
      ne_w2, ne_b2, ne_w3, ne_b3, ne_g, ne_b, wa0, wb0)

    ee_w1, ee_b1, ee_w2, ee_b2, ee_w3, ee_b3, ee_g, ee_b = _enc_w(params["edge_enc"])
    srcs = [src[sl * E2:(sl + 1) * E2] for sl in range(NSLAB)]
    dsts = [dst[sl * E2:(sl + 1) * E2] for sl in range(NSLAB)]
    es = [pl.pallas_call(
        _edge_encode_body,
        grid=(E2 // BE,),
        in_specs=[_row_spec(BE, 3)] + _wspecs(
            [(3, D), (1, D), (D, D), (1, D), (D, D), (1, D), (1, D), (1, D)]),
        out_specs=_row_spec(BE, D),
        out_shape=jax.ShapeDtypeStruct((E2, D), jnp.float32),
    )(edge_features[sl * E2:(sl + 1) * E2], ee_w1, ee_b1, ee_w2, ee_b2,
      ee_w3, ee_b3, ee_g, ee_b) for sl in range(NSLAB)]

    for s in range(NSTEPS):
        ef_w1, ef_b1, ef_w2, ef_b2, ef_w3, ef_b3, ef_g, ef_b = _enc_w(steps[s]["edge_fn"])
        aggs = []
        for sl in range(NSLAB):
            g = _gather_g(u, v, srcs[sl], dsts[sl])
            es[sl], enew = pl.pallas_call(
                _edge_step_body,
                grid=(E2 // BE,),
                in_specs=[_row_spec(BE, D), _row_spec(BE, D)] + _wspecs(
                    [(D, D), (1, D), (D, D), (1, D), (D, D), (1, D), (1, D), (1, D)]),
                out_specs=[_row_spec(BE, D)] * 2,
                out_shape=[jax.ShapeDtypeStruct((E2, D), jnp.float32)] * 2,
            )(g, es[sl], ef_w1[2 * D:3 * D], ef_b1, ef_w2, ef_b2, ef_w3,
              ef_b3, ef_g, ef_b)
            aggs.append(_scatter_agg(enew, dsts[sl])[:, :N, :])

        nf_w1, nf_b1, nf_w2, nf_b2, nf_w3, nf_b3, nf_g, nf_b = _enc_w(steps[s]["node_fn"])
        agg_spec = pl.BlockSpec((NC, BN, D), lambda i: (0, i, 0))
        if s < NSTEPS - 1:
            wa = steps[s + 1]["edge_fn"]["mlp"][0]["w"][0:D]
            wb = steps[s + 1]["edge_fn"]["mlp"][0]["w"][D:2 * D]
            x, u, v = pl.pallas_call(
                _node_step_body,
                grid=(N // BN,),
                in_specs=[_row_spec(BN, D)] + [agg_spec] * NSLAB + _wspecs(
                    [(D, D), (D, D), (1, D), (D, D), (1, D), (D, D), (1, D),
                     (1, D), (1, D), (D, D), (D, D)]),
                out_specs=[_row_spec(BN, D)] * 3,
                out_shape=[jax.ShapeDtypeStruct((N, D), jnp.float32)] * 3,
            )(x, *aggs, nf_w1[0:D], nf_w1[D:2 * D], nf_b1, nf_w2,
              nf_b2, nf_w3, nf_b3, nf_g, nf_b, wa, wb)
        else:
            dm = params["decoder"]["mlp"]
            out = pl.pallas_call(
                _final_body,
                grid=(N // BN,),
                in_specs=[_row_spec(BN, D)] + [agg_spec] * NSLAB + _wspecs(
                    [(D, D), (D, D), (1, D), (D, D), (1, D), (D, D), (1, D),
                     (1, D), (1, D),
                     (D, D), (1, D), (D, D), (1, D), (D, 3), (1, 3)]),
                out_specs=_row_spec(BN, 3),
                out_shape=jax.ShapeDtypeStruct((N, 3), jnp.float32),
            )(x, *aggs, nf_w1[0:D], nf_w1[D:2 * D], nf_b1, nf_w2,
              nf_b2, nf_w3, nf_b3, nf_g, nf_b,
              dm[0]["w"], dm[0]["b"].reshape(1, D),
              dm[1]["w"], dm[1]["b"].reshape(1, D),
              dm[2]["w"], dm[2]["b"].reshape(1, 3))
    return out
